# baseline (speedup 1.0000x reference)
import jax
import jax.numpy as jnp
from jax import lax
from jax.experimental import pallas as pl
from jax.experimental.pallas import tpu as pltpu
from jax.experimental.pallas import tpu_sc as plsc

_BATCH = 16384
_N_FIELDS = 26
_VOCAB = 100000
_EMB = 16

_NC = 2
_NS = 16
_NW = _NC * _NS
_L = 16

_ROWS_PER_W = _BATCH // _NW
_VECS = _ROWS_PER_W // _L


def _body(tab_hbm, x_hbm, out_hbm, cb_v, il0, il1, xv0, xv1, b0, b1,
          xs0, xs1, gs0, gs1, os0, os1):
    ils = (il0, il1)
    xvs = (xv0, xv1)
    bufs = (b0, b1)
    xsems = (xs0, xs1)
    gsems = (gs0, gs1)
    osems = (os0, os1)

    wid = lax.axis_index("s") * _NC + lax.axis_index("c")
    base = wid * _ROWS_PER_W

    iota = lax.iota(jnp.int32, _L)

    @pl.loop(0, _VECS)
    def _cb(k):
        cb_v[pl.ds(k * _L, _L)] = (iota + (base + k * _L)) * _N_FIELDS

    def fill_il(f, b):
        @pl.loop(0, _VECS)
        def _il(k):
            sl = pl.ds(k * _L, _L)
            ils[b][sl] = cb_v[sl] + f

    def xgather_desc(f, b):
        return pltpu.make_async_copy(x_hbm.at[ils[b]], xvs[b], xsems[b])

    def tgather_desc(f, b):
        return pltpu.make_async_copy(
            tab_hbm.at[f].at[xvs[b]], bufs[b], gsems[b])

    def out_desc(f, b):
        dst = out_hbm.at[pl.ds(base, _ROWS_PER_W), pl.ds(f * _EMB, _EMB)]
        return pltpu.make_async_copy(bufs[b], dst, osems[b])

    @pl.loop(0, _N_FIELDS // 2)
    def _fields(i):
        f0 = 2 * i
        f1 = f0 + 1

        @pl.when(i > 0)
        def _():
            out_desc(f0, 0).wait()
            out_desc(f1, 1).wait()

        fill_il(f0, 0)
        xgather_desc(f0, 0).start()
        fill_il(f1, 1)
        xgather_desc(f1, 1).start()

        xgather_desc(f0, 0).wait()
        tgather_desc(f0, 0).start()
        xgather_desc(f1, 1).wait()
        tgather_desc(f1, 1).start()

        tgather_desc(f0, 0).wait()
        out_desc(f0, 0).start()
        tgather_desc(f1, 1).wait()
        out_desc(f1, 1).start()

    out_desc(_N_FIELDS - 2, 0).wait()
    out_desc(_N_FIELDS - 1, 1).wait()


_gather_call = pl.kernel(
    _body,
    out_type=jax.ShapeDtypeStruct((_BATCH, _N_FIELDS * _EMB), jnp.float32),
    mesh=plsc.VectorSubcoreMesh(core_axis_name="c", subcore_axis_name="s",
                                num_cores=_NC, num_subcores=_NS),
    scratch_types=(
        [pltpu.VMEM((_ROWS_PER_W,), jnp.int32) for _ in range(5)]
        + [pltpu.VMEM((_ROWS_PER_W, _EMB), jnp.float32) for _ in range(2)]
        + [pltpu.SemaphoreType.DMA for _ in range(6)]
    ),
    compiler_params=pltpu.CompilerParams(use_tc_tiling_on_sc=False),
)


def kernel(x, tables):
    return _gather_call(tables, x.astype(jnp.int32).reshape(-1))


# speedup vs baseline: 1.0186x; 1.0186x over previous
import jax
import jax.numpy as jnp
from jax import lax
from jax.experimental import pallas as pl
from jax.experimental.pallas import tpu as pltpu
from jax.experimental.pallas import tpu_sc as plsc

_BATCH = 16384
_N_FIELDS = 26
_VOCAB = 100000
_EMB = 16

_NC = 2
_NS = 16
_NW = _NC * _NS
_L = 16

_TOTAL = _BATCH * _N_FIELDS
_ROWS_PER_W = _BATCH // _NW
_PER_W = _TOTAL // _NW
_ROWS_PER_CHUNK = 64
_CHUNK = _ROWS_PER_CHUNK * _N_FIELDS
_N_CHUNKS = _ROWS_PER_W // _ROWS_PER_CHUNK
_DMAS_PER_CHUNK = 2 * _ROWS_PER_CHUNK


def _body(tab_hbm, x_hbm, out_hbm, idx_v, b0, b1, g0, g1, o0, o1):
    bufs = (b0, b1)
    gsems = (g0, g1)
    osems = (o0, o1)

    wid = lax.axis_index("s") * _NC + lax.axis_index("c")
    base = wid * _PER_W

    pltpu.sync_copy(x_hbm.at[pl.ds(wid * _ROWS_PER_W, _ROWS_PER_W), :], idx_v)

    iota = lax.iota(jnp.int32, _L)
    pat_lo = iota * _VOCAB
    pat_hi = (iota + 10) * _VOCAB

    def fire_chunk(c, b):
        @pl.loop(0, _ROWS_PER_CHUNK)
        def _row(j):
            jg = c * _ROWS_PER_CHUNK + j
            v_lo = idx_v[jg, pl.ds(0, _L)] + pat_lo
            v_hi = idx_v[jg, pl.ds(_N_FIELDS - _L, _L)] + pat_hi
            pltpu.async_copy(
                tab_hbm.at[v_lo],
                bufs[b].at[pl.ds(j * _N_FIELDS, _L), :], gsems[b])
            pltpu.async_copy(
                tab_hbm.at[v_hi],
                bufs[b].at[pl.ds(j * _N_FIELDS + _N_FIELDS - _L, _L), :],
                gsems[b])

    def drain_chunk(b):
        @pl.loop(0, _DMAS_PER_CHUNK)
        def _w(k):
            pltpu.make_async_copy(
                tab_hbm.at[pl.ds(0, _L), :],
                bufs[b].at[pl.ds(0, _L), :], gsems[b]).wait()

    def out_desc(c, b):
        dst = out_hbm.at[pl.ds(base + c * _CHUNK, _CHUNK), :]
        return pltpu.make_async_copy(bufs[b], dst, osems[b])

    @pl.loop(0, _N_CHUNKS // 2)
    def _chunks(i):
        c0 = 2 * i
        c1 = c0 + 1

        @pl.when(i > 0)
        def _():
            out_desc(c0, 0).wait()
            out_desc(c1, 1).wait()

        fire_chunk(c0, 0)
        fire_chunk(c1, 1)
        drain_chunk(0)
        out_desc(c0, 0).start()
        drain_chunk(1)
        out_desc(c1, 1).start()

    out_desc(_N_CHUNKS - 2, 0).wait()
    out_desc(_N_CHUNKS - 1, 1).wait()


_gather_call = pl.kernel(
    _body,
    out_type=jax.ShapeDtypeStruct((_TOTAL, _EMB), jnp.float32),
    mesh=plsc.VectorSubcoreMesh(core_axis_name="c", subcore_axis_name="s",
                                num_cores=_NC, num_subcores=_NS),
    scratch_types=(
        [pltpu.VMEM((_ROWS_PER_W, _N_FIELDS), jnp.int32)]
        + [pltpu.VMEM((_CHUNK, _EMB), jnp.float32) for _ in range(2)]
        + [pltpu.SemaphoreType.DMA for _ in range(4)]
    ),
    compiler_params=pltpu.CompilerParams(use_tc_tiling_on_sc=False),
)


def kernel(x, tables):
    tab = tables.reshape(_N_FIELDS * _VOCAB, _EMB)
    out = _gather_call(tab, x.astype(jnp.int32))
    return out.reshape(_BATCH, _N_FIELDS * _EMB)
